# baseline (speedup 1.0000x reference)
import functools

import jax
import jax.numpy as jnp
from jax import lax
from jax.experimental import pallas as pl
from jax.experimental.pallas import tpu as pltpu
from jax.experimental.pallas import tpu_sc as plsc

B, S = 4096, 200
D = 64
TOTAL = B * S
NC, NS = 2, 16
NW = NC * NS
PER_W = TOTAL // NW
CHUNK = 128
N_CHUNKS = PER_W // CHUNK

_mesh = plsc.VectorSubcoreMesh(core_axis_name="c", subcore_axis_name="s")


@functools.partial(
    pl.kernel,
    out_type=jax.ShapeDtypeStruct((TOTAL, D), jnp.float32),
    mesh=_mesh,
    scratch_types=[
        pltpu.VMEM((N_CHUNKS, CHUNK), jnp.int32),
        pltpu.VMEM((2, CHUNK, D), jnp.float32),
        pltpu.SemaphoreType.DMA((2,)),
        pltpu.SemaphoreType.DMA((2,)),
    ],
    compiler_params=pltpu.CompilerParams(use_tc_tiling_on_sc=False),
)
def _gather_kernel(idx_hbm, table_hbm, out_hbm, idx_v, rows_v, gsem, ssem):
    wid = lax.axis_index("s") * NC + lax.axis_index("c")
    row_base = wid * N_CHUNKS

    pltpu.sync_copy(idx_hbm.at[pl.ds(row_base, N_CHUNKS)], idx_v)

    def gather(j, buf):
        return pltpu.async_copy(
            table_hbm.at[idx_v.at[j]], rows_v.at[buf], gsem.at[buf]
        )

    def store(j, buf):
        return pltpu.async_copy(
            rows_v.at[buf],
            out_hbm.at[pl.ds((row_base + j) * CHUNK, CHUNK)],
            ssem.at[buf],
        )

    gather(0, 0).wait()
    store(0, 0)
    gather(1, 1)

    def body(j, _):
        buf = lax.rem(j, 2)
        nxt = 1 - buf
        pltpu.make_async_copy(
            table_hbm.at[idx_v.at[j]], rows_v.at[buf], gsem.at[buf]
        ).wait()
        store(j, buf)
        pltpu.make_async_copy(
            rows_v.at[nxt],
            out_hbm.at[pl.ds((row_base + j - 1) * CHUNK, CHUNK)],
            ssem.at[nxt],
        ).wait()

        @pl.when(j + 1 < N_CHUNKS)
        def _():
            gather(j + 1, nxt)

        return 0

    lax.fori_loop(1, N_CHUNKS, body, 0)

    last = N_CHUNKS - 1
    pltpu.make_async_copy(
        rows_v.at[last % 2],
        out_hbm.at[pl.ds((row_base + last) * CHUNK, CHUNK)],
        ssem.at[last % 2],
    ).wait()


def kernel(zone_ids, table):
    idx = zone_ids.reshape(TOTAL // CHUNK, CHUNK).astype(jnp.int32)
    out = _gather_kernel(idx, table)
    return out.reshape(B, S, D)


# speedup vs baseline: 1.0712x; 1.0712x over previous
import functools

import jax
import jax.numpy as jnp
from jax import lax
from jax.experimental import pallas as pl
from jax.experimental.pallas import tpu as pltpu
from jax.experimental.pallas import tpu_sc as plsc

B, S = 4096, 200
D = 64
TOTAL = B * S
NC, NS = 2, 16
NW = NC * NS
PER_W = TOTAL // NW
CHUNK = 128
N_CHUNKS = PER_W // CHUNK
NBUF = 8
GROUPS = N_CHUNKS // NBUF

_mesh = plsc.VectorSubcoreMesh(core_axis_name="c", subcore_axis_name="s")


@functools.partial(
    pl.kernel,
    out_type=jax.ShapeDtypeStruct((TOTAL, D), jnp.float32),
    mesh=_mesh,
    scratch_types=[
        pltpu.VMEM((N_CHUNKS, CHUNK), jnp.int32),
        pltpu.VMEM((NBUF, CHUNK, D), jnp.float32),
        pltpu.SemaphoreType.DMA((NBUF,)),
        pltpu.SemaphoreType.DMA((NBUF,)),
    ],
    compiler_params=pltpu.CompilerParams(use_tc_tiling_on_sc=False),
)
def _gather_kernel(idx_hbm, table_hbm, out_hbm, idx_v, rows_v, gsem, ssem):
    wid = lax.axis_index("s") * NC + lax.axis_index("c")
    row_base = wid * N_CHUNKS

    pltpu.sync_copy(idx_hbm.at[pl.ds(row_base, N_CHUNKS)], idx_v)

    def gather(j, buf):
        return pltpu.async_copy(
            table_hbm.at[idx_v.at[j]], rows_v.at[buf], gsem.at[buf]
        )

    def store(j, buf):
        return pltpu.async_copy(
            rows_v.at[buf],
            out_hbm.at[pl.ds((row_base + j) * CHUNK, CHUNK)],
            ssem.at[buf],
        )

    for b in range(NBUF):
        gather(b, b)

    def group(g, _):
        base = g * NBUF
        for b in range(NBUF):
            j = base + b
            pltpu.make_async_copy(
                table_hbm.at[idx_v.at[j]], rows_v.at[b], gsem.at[b]
            ).wait()
            store(j, b)
        for b in range(NBUF):
            j = base + b
            pltpu.make_async_copy(
                rows_v.at[b],
                out_hbm.at[pl.ds((row_base + j) * CHUNK, CHUNK)],
                ssem.at[b],
            ).wait()

            @pl.when(j + NBUF < N_CHUNKS)
            def _():
                gather(j + NBUF, b)

        return 0

    lax.fori_loop(0, GROUPS, group, 0)


def kernel(zone_ids, table):
    idx = zone_ids.reshape(TOTAL // CHUNK, CHUNK).astype(jnp.int32)
    out = _gather_kernel(idx, table)
    return out.reshape(B, S, D)
